# Initial kernel scaffold; baseline (speedup 1.0000x reference)
import jax
import jax.numpy as jnp
from jax.experimental import pallas as pl


def kernel(x, Wq, Wk, Wv, Wskip):
    raise NotImplementedError("write your pallas kernel here")


# speedup vs baseline: 97.6426x; 97.6426x over previous
import jax
import jax.numpy as jnp
from jax.experimental import pallas as pl
from jax.experimental.pallas import tpu as pltpu

_B, _N, _C = 4, 2048, 256
_H, _D = 8, 64
_HD = _H * _D
_K = 32
_TR = 256
_PR = 512


def _proj_kernel(x_ref, wq_ref, wk_ref, wv_ref, ws_ref,
                 q_ref, k_ref, v_ref, s_ref):
    x = x_ref[...]
    q_ref[...] = jnp.dot(x, wq_ref[...], preferred_element_type=jnp.float32)
    k_ref[...] = jnp.dot(x, wk_ref[...], preferred_element_type=jnp.float32)
    v_ref[...] = jnp.dot(x, wv_ref[...], preferred_element_type=jnp.float32)
    s_ref[...] = jnp.dot(x, ws_ref[...], preferred_element_type=jnp.float32)


def _attn_kernel(xr_ref, xb_ref, q_ref, k_ref, v_ref, skip_ref, o_ref,
                 s_scr, scur_scr):
    rt = pl.program_id(1)
    xr = xr_ref[0]
    xb = xb_ref[0]
    sim = jax.lax.dot_general(xr, xb, (((1,), (1,)), ((), ())),
                              preferred_element_type=jnp.float32)
    rows = rt * _TR + jax.lax.broadcasted_iota(jnp.int32, (_TR, _N), 0)
    cols = jax.lax.broadcasted_iota(jnp.int32, (_TR, _N), 1)
    sim = sim - jnp.where(rows == cols, 1e9, 0.0).astype(jnp.float32)
    s_scr[...] = sim
    scur_scr[...] = sim

    def body(i, carry):
        sc = scur_scr[...]
        m = jnp.max(sc, axis=1, keepdims=True)
        scur_scr[...] = jnp.where(sc >= m, -jnp.inf, sc)
        return carry

    jax.lax.fori_loop(0, _K - 1, body, 0)
    thresh = jnp.max(scur_scr[...], axis=1, keepdims=True)
    mask = s_scr[...] >= thresh

    skip = skip_ref[0]
    inv_sqrt_d = 1.0 / (_D ** 0.5)
    for h in range(_H):
        sl = slice(h * _D, (h + 1) * _D)
        qh = q_ref[0][:, sl]
        kh = k_ref[0][:, sl]
        vh = v_ref[0][:, sl]
        logits = jax.lax.dot_general(qh, kh, (((1,), (1,)), ((), ())),
                                     preferred_element_type=jnp.float32)
        logits = jnp.where(mask, logits * inv_sqrt_d, -jnp.inf)
        m = jnp.max(logits, axis=1, keepdims=True)
        e = jnp.where(mask, jnp.exp(logits - m), 0.0)
        ssum = jnp.sum(e, axis=1, keepdims=True)
        alpha = e / (ssum + 1e-16)
        oh = jnp.dot(alpha, vh, preferred_element_type=jnp.float32)
        o_ref[0, :, sl] = oh + skip[:, sl]


def _project(xf, Wq, Wk, Wv, Wskip):
    grid = (_B * _N // _PR,)
    wspec = pl.BlockSpec((_C, _HD), lambda i: (0, 0))
    rspec = pl.BlockSpec((_PR, _HD), lambda i: (i, 0))
    return pl.pallas_call(
        _proj_kernel,
        grid=grid,
        in_specs=[pl.BlockSpec((_PR, _C), lambda i: (i, 0)),
                  wspec, wspec, wspec, wspec],
        out_specs=[rspec, rspec, rspec, rspec],
        out_shape=[jax.ShapeDtypeStruct((_B * _N, _HD), jnp.float32)] * 4,
    )(xf, Wq, Wk, Wv, Wskip)


def _attention(x, q, k, v, skip):
    grid = (_B, _N // _TR)
    row3 = pl.BlockSpec((1, _TR, _HD), lambda b, r: (b, r, 0))
    full3 = pl.BlockSpec((1, _N, _HD), lambda b, r: (b, 0, 0))
    return pl.pallas_call(
        _attn_kernel,
        grid=grid,
        in_specs=[pl.BlockSpec((1, _TR, _C), lambda b, r: (b, r, 0)),
                  pl.BlockSpec((1, _N, _C), lambda b, r: (b, 0, 0)),
                  row3, full3, full3, row3],
        out_specs=row3,
        out_shape=jax.ShapeDtypeStruct((_B, _N, _HD), jnp.float32),
        scratch_shapes=[pltpu.VMEM((_TR, _N), jnp.float32),
                        pltpu.VMEM((_TR, _N), jnp.float32)],
    )(x, x, q, k, v, skip)


def kernel(x, Wq, Wk, Wv, Wskip):
    xf = x.reshape(_B * _N, _C)
    q, k, v, skip = _project(xf, Wq, Wk, Wv, Wskip)
    q = q.reshape(_B, _N, _HD)
    k = k.reshape(_B, _N, _HD)
    v = v.reshape(_B, _N, _HD)
    skip = skip.reshape(_B, _N, _HD)
    return _attention(x, q, k, v, skip)
